# baseline (speedup 1.0000x reference)
import functools

import jax
import jax.numpy as jnp
from jax import lax
from jax.experimental import pallas as pl
from jax.experimental.pallas import tpu as pltpu
from jax.experimental.pallas import tpu_sc as plsc

NC = 2
NS = 16
LANES = 16

K = 128
DEG_W = 16


def _mo8(v):
  return pl.multiple_of(v, 8)


def _chunk_plan(total, step):
  full, rem = divmod(total, step)
  plan = [(z * step, step) for z in range(full)]
  if rem:
    plan.append((full * step, rem))
  return plan


def _fill_f32(ref, rows, value):
  ncols = ref.shape[1]
  vec = jnp.full((LANES,), value, jnp.float32)

  def body(i, _):
    for j in range(ncols // LANES):
      ref[i, pl.ds(j * LANES, LANES)] = vec
    return 0

  lax.fori_loop(0, rows, body, 0)


def _deg_body(dst_hbm, out_hbm, idx_v, vals_v, deg_acc, sem, *, e_pad, n_acc):
  c = lax.axis_index("c")
  s = lax.axis_index("s")
  wid = c * NS + s

  rows_per_tile = n_acc // NS
  r0 = s * rows_per_tile
  _fill_f32(vals_v, K, 0.0)
  for z0, zn in _chunk_plan(rows_per_tile, K):
    pltpu.sync_copy(vals_v.at[pl.ds(0, zn)], deg_acc.at[pl.ds(_mo8(r0 + z0), zn)])
  _fill_f32(vals_v, K, 1.0)
  plsc.subcore_barrier()

  edges_per_worker = e_pad // (NC * NS)
  nchunks = edges_per_worker // K
  base = wid * edges_per_worker

  def chunk(i, _):
    pltpu.sync_copy(dst_hbm.at[pl.ds(_mo8(base + i * K), K)], idx_v)
    pltpu.sync_copy(vals_v, deg_acc.at[idx_v], add=True)
    return 0

  lax.fori_loop(0, nchunks, chunk, 0)
  plsc.subcore_barrier()

  for z0, zn in _chunk_plan(rows_per_tile, K):
    pltpu.sync_copy(deg_acc.at[pl.ds(_mo8(r0 + z0), zn)],
                    out_hbm.at[c, pl.ds(_mo8(r0 + z0), zn)])


def _agg_body(y_hbm, src_hbm, dst_hbm, out_hbm, src_v, dst_v, rows_v, acc,
              sem, *, e_pad, n_acc, d, t_per_core):
  c = lax.axis_index("c")
  s = lax.axis_index("s")

  rows_per_tile = n_acc // NS
  edges_per_tile = e_pad // NS
  nchunks = edges_per_tile // K
  ebase = s * edges_per_tile

  src_v = tuple(src_v)
  dst_v = tuple(dst_v)
  rows_v = tuple(rows_v)

  def sync_idx(t_idx, i, slot):
    e1 = t_idx * e_pad + ebase + i * K
    pltpu.sync_copy(src_hbm.at[pl.ds(_mo8(e1), K)], src_v[slot])
    pltpu.sync_copy(dst_hbm.at[pl.ds(_mo8(ebase + i * K), K)], dst_v[slot])

  for tt in range(t_per_core):
    t_idx = c * t_per_core + tt
    _fill_f32(rows_v[0], K, 0.0)
    r0 = s * rows_per_tile
    for z0, zn in _chunk_plan(rows_per_tile, K):
      pltpu.sync_copy(rows_v[0].at[pl.ds(0, zn)], acc.at[pl.ds(_mo8(r0 + z0), zn)])
    plsc.subcore_barrier()

    def do_chunk(i, slot, first, has_next):
      nxt = (slot + 1) % 3
      nx2 = (slot + 2) % 3
      if has_next:
        sync_idx(t_idx, i + 1, nxt)
      pltpu.make_async_copy(y_hbm.at[src_v[slot]], rows_v[slot],
                            sem).wait()
      if has_next:
        pltpu.async_copy(y_hbm.at[src_v[nxt]], rows_v[nxt], sem)
      pltpu.sync_copy(rows_v[slot], acc.at[dst_v[slot]], add=True)

    sync_idx(t_idx, 0, 0)
    pltpu.async_copy(y_hbm.at[src_v[0]], rows_v[0], sem)

    do_chunk(0, 0, True, True)
    do_chunk(1, 1, False, True)
    do_chunk(2, 2, False, True)

    def outer(iu, _):
      i = iu * 3
      do_chunk(i, 0, False, True)
      do_chunk(i + 1, 1, False, True)
      do_chunk(i + 2, 2, False, True)
      return 0

    lax.fori_loop(1, nchunks // 3 - 1, outer, 0)
    i = nchunks - 3
    do_chunk(i, 0, False, True)
    do_chunk(i + 1, 1, False, True)
    do_chunk(i + 2, 2, False, False)
    plsc.subcore_barrier()

    for z0, zn in _chunk_plan(rows_per_tile, K):
      pltpu.sync_copy(acc.at[pl.ds(_mo8(r0 + z0), zn)],
                      out_hbm.at[t_idx, pl.ds(_mo8(r0 + z0), zn)])
    plsc.subcore_barrier()


def _matmul_body(x_ref, w_ref, degp_ref, y_ref, dinv_ref):
  dinv = lax.rsqrt(degp_ref[0, :, 0:1] + degp_ref[1, :, 0:1] + 1.0)
  dinv_ref[0] = dinv
  xw = jnp.dot(x_ref[0], w_ref[...], preferred_element_type=jnp.float32)
  y_ref[0] = xw * dinv


def _epilogue_body(agg_ref, y_ref, dinv_ref, b_ref, out_ref):
  out_ref[0] = dinv_ref[0] * (agg_ref[0] + y_ref[0]) + b_ref[...]


def kernel(x, edge_index, W, b):
  n, d, t_window = x.shape
  e = edge_index.shape[1]
  t_per_core = t_window // NC

  unit = NC * NS * K * 3
  e_pad = ((e + unit - 1) // unit) * unit
  n_acc = ((n + NS * 8) // (NS * 8)) * (NS * 8)
  pad = e_pad - e
  src = jnp.concatenate([edge_index[0], jnp.zeros((pad,), jnp.int32)])
  dst = jnp.concatenate([edge_index[1], jnp.full((pad,), n, jnp.int32)])
  src_adj = (src[None, :]
             + (jnp.arange(t_window, dtype=jnp.int32) * n)[:, None]).reshape(-1)

  mesh = plsc.VectorSubcoreMesh(core_axis_name="c", subcore_axis_name="s")

  deg_kernel = pl.kernel(
      functools.partial(_deg_body, e_pad=e_pad, n_acc=n_acc),
      out_type=jax.ShapeDtypeStruct((NC, n_acc, DEG_W), jnp.float32),
      mesh=mesh,
      scratch_types=[
          pltpu.VMEM((K,), jnp.int32),
          pltpu.VMEM((K, DEG_W), jnp.float32),
          pltpu.VMEM_SHARED((n_acc, DEG_W), jnp.float32),
          pltpu.SemaphoreType.DMA,
      ],
  )
  deg_parts = deg_kernel(dst)

  xt = x.transpose(2, 0, 1).reshape(t_window * n, d)
  bn = 400
  nb = n // bn
  y3d, dinv = pl.pallas_call(
      _matmul_body,
      grid=(t_window, nb),
      in_specs=[
          pl.BlockSpec((1, bn, d), lambda t, i: (t, i, 0)),
          pl.BlockSpec((d, d), lambda t, i: (0, 0)),
          pl.BlockSpec((NC, bn, DEG_W), lambda t, i: (0, i, 0)),
      ],
      out_specs=[
          pl.BlockSpec((1, bn, d), lambda t, i: (t, i, 0)),
          pl.BlockSpec((1, bn, 1), lambda t, i: (t, i, 0)),
      ],
      out_shape=[
          jax.ShapeDtypeStruct((t_window, n, d), jnp.float32),
          jax.ShapeDtypeStruct((t_window, n, 1), jnp.float32),
      ],
  )(xt.reshape(t_window, n, d), W, deg_parts)
  y_flat = y3d.reshape(t_window * n, d)

  agg_kernel = pl.kernel(
      functools.partial(_agg_body, e_pad=e_pad, n_acc=n_acc, d=d,
                        t_per_core=t_per_core),
      out_type=jax.ShapeDtypeStruct((t_window, n_acc, d), jnp.float32),
      mesh=mesh,
      scratch_types=[
          [pltpu.VMEM((K,), jnp.int32) for _ in range(3)],
          [pltpu.VMEM((K,), jnp.int32) for _ in range(3)],
          [pltpu.VMEM((K, d), jnp.float32) for _ in range(3)],
          pltpu.VMEM_SHARED((n_acc, d), jnp.float32),
          pltpu.SemaphoreType.DMA,
      ],
  )
  agg = agg_kernel(y_flat, src_adj, dst)

  out = pl.pallas_call(
      _epilogue_body,
      grid=(t_window, nb),
      in_specs=[
          pl.BlockSpec((1, bn, d), lambda t, i: (t, i, 0)),
          pl.BlockSpec((1, bn, d), lambda t, i: (t, i, 0)),
          pl.BlockSpec((1, bn, 1), lambda t, i: (t, i, 0)),
          pl.BlockSpec((1, d), lambda t, i: (0, 0)),
      ],
      out_specs=pl.BlockSpec((1, bn, d), lambda t, i: (t, i, 0)),
      out_shape=jax.ShapeDtypeStruct((t_window, n, d), jnp.float32),
  )(agg, y3d, dinv, b.reshape(1, d))
  return out.transpose(1, 2, 0)


# speedup vs baseline: 1.8150x; 1.8150x over previous
import functools

import jax
import jax.numpy as jnp
from jax import lax
from jax.experimental import pallas as pl
from jax.experimental.pallas import tpu as pltpu
from jax.experimental.pallas import tpu_sc as plsc

NC = 2
NS = 16
LANES = 16

K = 112
DEG_W = 16


def _mo8(v):
  return pl.multiple_of(v, 8)


def _chunk_plan(total, step):
  full, rem = divmod(total, step)
  plan = [(z * step, step) for z in range(full)]
  if rem:
    plan.append((full * step, rem))
  return plan


def _fill_f32(ref, rows, value):
  ncols = ref.shape[1]
  vec = jnp.full((LANES,), value, jnp.float32)

  def body(i, _):
    for j in range(ncols // LANES):
      ref[i, pl.ds(j * LANES, LANES)] = vec
    return 0

  lax.fori_loop(0, rows, body, 0)


def _deg_body(dst_hbm, out_hbm, idx_v, vals_v, deg_acc, sem, *, e_pad, n_acc):
  c = lax.axis_index("c")
  s = lax.axis_index("s")
  wid = c * NS + s

  rows_per_tile = n_acc // NS
  r0 = s * rows_per_tile
  _fill_f32(vals_v, K, 0.0)
  for z0, zn in _chunk_plan(rows_per_tile, K):
    pltpu.sync_copy(vals_v.at[pl.ds(0, zn)], deg_acc.at[pl.ds(_mo8(r0 + z0), zn)])
  _fill_f32(vals_v, K, 1.0)
  plsc.subcore_barrier()

  edges_per_worker = e_pad // (NC * NS)
  nchunks = edges_per_worker // K
  base = wid * edges_per_worker

  def chunk(i, _):
    pltpu.sync_copy(dst_hbm.at[pl.ds(_mo8(base + i * K), K)], idx_v)
    pltpu.sync_copy(vals_v, deg_acc.at[idx_v], add=True)
    return 0

  lax.fori_loop(0, nchunks, chunk, 0)
  plsc.subcore_barrier()

  for z0, zn in _chunk_plan(rows_per_tile, K):
    pltpu.sync_copy(deg_acc.at[pl.ds(_mo8(r0 + z0), zn)],
                    out_hbm.at[c, pl.ds(_mo8(r0 + z0), zn)])


def _agg_body(y_hbm, src_hbm, dst_hbm, out_hbm, src_v, dst_v, rows_v, acc,
              sem, *, e_pad, n_acc, d, t_per_core):
  c = lax.axis_index("c")
  s = lax.axis_index("s")

  rows_per_tile = n_acc // NS
  edges_per_tile = e_pad // NS
  nchunks = edges_per_tile // K
  ebase = s * edges_per_tile

  src_v = tuple(src_v)
  dst_v = tuple(dst_v)
  rows_v = tuple(rows_v)

  def sync_idx(t_idx, i, slot):
    e1 = t_idx * e_pad + ebase + i * K
    pltpu.sync_copy(src_hbm.at[pl.ds(_mo8(e1), K)], src_v[slot])
    pltpu.sync_copy(dst_hbm.at[pl.ds(_mo8(ebase + i * K), K)], dst_v[slot])

  for tt in range(t_per_core):
    t_idx = c * t_per_core + tt
    _fill_f32(rows_v[0], K, 0.0)
    r0 = s * rows_per_tile
    for z0, zn in _chunk_plan(rows_per_tile, K):
      pltpu.sync_copy(rows_v[0].at[pl.ds(0, zn)], acc.at[pl.ds(_mo8(r0 + z0), zn)])
    plsc.subcore_barrier()

    def do_chunk(i, slot, first, has_next):
      nxt = (slot + 1) % 3
      nx2 = (slot + 2) % 3
      if has_next:
        sync_idx(t_idx, i + 1, nxt)
      pltpu.make_async_copy(y_hbm.at[src_v[slot]], rows_v[slot],
                            sem).wait()
      if has_next:
        pltpu.async_copy(y_hbm.at[src_v[nxt]], rows_v[nxt], sem)
      pltpu.sync_copy(rows_v[slot], acc.at[dst_v[slot]], add=True)

    sync_idx(t_idx, 0, 0)
    pltpu.async_copy(y_hbm.at[src_v[0]], rows_v[0], sem)

    do_chunk(0, 0, True, True)
    do_chunk(1, 1, False, True)
    do_chunk(2, 2, False, True)

    def outer(iu, _):
      i = iu * 3
      do_chunk(i, 0, False, True)
      do_chunk(i + 1, 1, False, True)
      do_chunk(i + 2, 2, False, True)
      return 0

    lax.fori_loop(1, nchunks // 3 - 1, outer, 0)
    i = nchunks - 3
    do_chunk(i, 0, False, True)
    do_chunk(i + 1, 1, False, True)
    do_chunk(i + 2, 2, False, False)
    plsc.subcore_barrier()

    for z0, zn in _chunk_plan(rows_per_tile, K):
      pltpu.sync_copy(acc.at[pl.ds(_mo8(r0 + z0), zn)],
                      out_hbm.at[t_idx, pl.ds(_mo8(r0 + z0), zn)])
    plsc.subcore_barrier()


def _matmul_body(x_ref, w_ref, degp_ref, y_ref, dinv_ref):
  dinv = lax.rsqrt(degp_ref[0, :, 0:1] + degp_ref[1, :, 0:1] + 1.0)
  dinv_ref[0] = dinv
  xw = jnp.dot(x_ref[0], w_ref[...], preferred_element_type=jnp.float32)
  y_ref[0] = xw * dinv


def _epilogue_body(agg_ref, y_ref, dinv_ref, b_ref, out_ref):
  out_ref[0] = dinv_ref[0] * (agg_ref[0] + y_ref[0]) + b_ref[...]


def kernel(x, edge_index, W, b):
  n, d, t_window = x.shape
  e = edge_index.shape[1]
  t_per_core = t_window // NC

  unit = NC * NS * K * 3
  e_pad = ((e + unit - 1) // unit) * unit
  n_acc = ((n + NS * 8) // (NS * 8)) * (NS * 8)
  pad = e_pad - e
  src = jnp.concatenate([edge_index[0], jnp.zeros((pad,), jnp.int32)])
  dst = jnp.concatenate([edge_index[1], jnp.full((pad,), n, jnp.int32)])
  src_adj = (src[None, :]
             + (jnp.arange(t_window, dtype=jnp.int32) * n)[:, None]).reshape(-1)

  mesh = plsc.VectorSubcoreMesh(core_axis_name="c", subcore_axis_name="s")

  deg_kernel = pl.kernel(
      functools.partial(_deg_body, e_pad=e_pad, n_acc=n_acc),
      out_type=jax.ShapeDtypeStruct((NC, n_acc, DEG_W), jnp.float32),
      mesh=mesh,
      scratch_types=[
          pltpu.VMEM((K,), jnp.int32),
          pltpu.VMEM((K, DEG_W), jnp.float32),
          pltpu.VMEM_SHARED((n_acc, DEG_W), jnp.float32),
          pltpu.SemaphoreType.DMA,
      ],
  )
  deg_parts = deg_kernel(dst)

  xt = x.transpose(2, 0, 1).reshape(t_window * n, d)
  bn = 400
  nb = n // bn
  y3d, dinv = pl.pallas_call(
      _matmul_body,
      grid=(t_window, nb),
      in_specs=[
          pl.BlockSpec((1, bn, d), lambda t, i: (t, i, 0)),
          pl.BlockSpec((d, d), lambda t, i: (0, 0)),
          pl.BlockSpec((NC, bn, DEG_W), lambda t, i: (0, i, 0)),
      ],
      out_specs=[
          pl.BlockSpec((1, bn, d), lambda t, i: (t, i, 0)),
          pl.BlockSpec((1, bn, 1), lambda t, i: (t, i, 0)),
      ],
      out_shape=[
          jax.ShapeDtypeStruct((t_window, n, d), jnp.float32),
          jax.ShapeDtypeStruct((t_window, n, 1), jnp.float32),
      ],
  )(xt.reshape(t_window, n, d), W, deg_parts)
  y_flat = y3d.reshape(t_window * n, d)

  agg_kernel = pl.kernel(
      functools.partial(_agg_body, e_pad=e_pad, n_acc=n_acc, d=d,
                        t_per_core=t_per_core),
      out_type=jax.ShapeDtypeStruct((t_window, n_acc, d), jnp.float32),
      mesh=mesh,
      scratch_types=[
          [pltpu.VMEM((K,), jnp.int32) for _ in range(3)],
          [pltpu.VMEM((K,), jnp.int32) for _ in range(3)],
          [pltpu.VMEM((K, d), jnp.float32) for _ in range(3)],
          pltpu.VMEM_SHARED((n_acc, d), jnp.float32),
          pltpu.SemaphoreType.DMA,
      ],
  )
  agg = agg_kernel(y_flat, src_adj, dst)

  out = pl.pallas_call(
      _epilogue_body,
      grid=(t_window, nb),
      in_specs=[
          pl.BlockSpec((1, bn, d), lambda t, i: (t, i, 0)),
          pl.BlockSpec((1, bn, d), lambda t, i: (t, i, 0)),
          pl.BlockSpec((1, bn, 1), lambda t, i: (t, i, 0)),
          pl.BlockSpec((1, d), lambda t, i: (0, 0)),
      ],
      out_specs=pl.BlockSpec((1, bn, d), lambda t, i: (t, i, 0)),
      out_shape=jax.ShapeDtypeStruct((t_window, n, d), jnp.float32),
  )(agg, y3d, dinv, b.reshape(1, d))
  return out.transpose(1, 2, 0)


# speedup vs baseline: 1.8450x; 1.0165x over previous
import functools

import jax
import jax.numpy as jnp
from jax import lax
from jax.experimental import pallas as pl
from jax.experimental.pallas import tpu as pltpu
from jax.experimental.pallas import tpu_sc as plsc

NC = 2
NS = 16
LANES = 16

K = 120
DEG_W = 16


def _mo8(v):
  return pl.multiple_of(v, 8)


def _chunk_plan(total, step):
  full, rem = divmod(total, step)
  plan = [(z * step, step) for z in range(full)]
  if rem:
    plan.append((full * step, rem))
  return plan


def _fill_f32(ref, rows, value):
  ncols = ref.shape[1]
  vec = jnp.full((LANES,), value, jnp.float32)

  def body(i, _):
    for j in range(ncols // LANES):
      ref[i, pl.ds(j * LANES, LANES)] = vec
    return 0

  lax.fori_loop(0, rows, body, 0)


def _deg_body(dst_hbm, out_hbm, idx_v, vals_v, deg_acc, sem, *, e_pad, n_acc):
  c = lax.axis_index("c")
  s = lax.axis_index("s")
  wid = c * NS + s

  rows_per_tile = n_acc // NS
  r0 = s * rows_per_tile
  _fill_f32(vals_v, K, 0.0)
  for z0, zn in _chunk_plan(rows_per_tile, K):
    pltpu.sync_copy(vals_v.at[pl.ds(0, zn)], deg_acc.at[pl.ds(_mo8(r0 + z0), zn)])
  _fill_f32(vals_v, K, 1.0)
  plsc.subcore_barrier()

  edges_per_worker = e_pad // (NC * NS)
  nchunks = edges_per_worker // K
  base = wid * edges_per_worker

  def chunk(i, _):
    pltpu.sync_copy(dst_hbm.at[pl.ds(_mo8(base + i * K), K)], idx_v)
    pltpu.sync_copy(vals_v, deg_acc.at[idx_v], add=True)
    return 0

  lax.fori_loop(0, nchunks, chunk, 0)
  plsc.subcore_barrier()

  for z0, zn in _chunk_plan(rows_per_tile, K):
    pltpu.sync_copy(deg_acc.at[pl.ds(_mo8(r0 + z0), zn)],
                    out_hbm.at[c, pl.ds(_mo8(r0 + z0), zn)])


def _agg_body(y_hbm, src_hbm, dst_hbm, out_hbm, src_v, dst_v, rows_v, acc,
              sem, *, e_pad, n_acc, d, t_per_core):
  c = lax.axis_index("c")
  s = lax.axis_index("s")

  rows_per_tile = n_acc // NS
  edges_per_tile = e_pad // NS
  nchunks = edges_per_tile // K
  ebase = s * edges_per_tile

  src_v = tuple(src_v)
  dst_v = tuple(dst_v)
  rows_v = tuple(rows_v)

  def sync_idx(t_idx, i, slot):
    e1 = t_idx * e_pad + ebase + i * K
    pltpu.sync_copy(src_hbm.at[pl.ds(_mo8(e1), K)], src_v[slot])
    pltpu.sync_copy(dst_hbm.at[pl.ds(_mo8(ebase + i * K), K)], dst_v[slot])

  for tt in range(t_per_core):
    t_idx = c * t_per_core + tt
    _fill_f32(rows_v[0], K, 0.0)
    r0 = s * rows_per_tile
    for z0, zn in _chunk_plan(rows_per_tile, K):
      pltpu.sync_copy(rows_v[0].at[pl.ds(0, zn)], acc.at[pl.ds(_mo8(r0 + z0), zn)])
    plsc.subcore_barrier()

    def do_chunk(i, slot, first, has_next):
      nxt = (slot + 1) % 3
      nx2 = (slot + 2) % 3
      if has_next:
        sync_idx(t_idx, i + 1, nxt)
      pltpu.make_async_copy(y_hbm.at[src_v[slot]], rows_v[slot],
                            sem).wait()
      if has_next:
        pltpu.async_copy(y_hbm.at[src_v[nxt]], rows_v[nxt], sem)
      pltpu.sync_copy(rows_v[slot], acc.at[dst_v[slot]], add=True)

    sync_idx(t_idx, 0, 0)
    pltpu.async_copy(y_hbm.at[src_v[0]], rows_v[0], sem)

    do_chunk(0, 0, True, True)
    do_chunk(1, 1, False, True)
    do_chunk(2, 2, False, True)

    def outer(iu, _):
      i = iu * 3
      do_chunk(i, 0, False, True)
      do_chunk(i + 1, 1, False, True)
      do_chunk(i + 2, 2, False, True)
      return 0

    lax.fori_loop(1, nchunks // 3 - 1, outer, 0)
    i = nchunks - 3
    do_chunk(i, 0, False, True)
    do_chunk(i + 1, 1, False, True)
    do_chunk(i + 2, 2, False, False)
    plsc.subcore_barrier()

    for z0, zn in _chunk_plan(rows_per_tile, K):
      pltpu.sync_copy(acc.at[pl.ds(_mo8(r0 + z0), zn)],
                      out_hbm.at[t_idx, pl.ds(_mo8(r0 + z0), zn)])
    plsc.subcore_barrier()


def _matmul_body(x_ref, w_ref, degp_ref, y_ref, dinv_ref):
  dinv = lax.rsqrt(degp_ref[0, :, 0:1] + degp_ref[1, :, 0:1] + 1.0)
  dinv_ref[0] = dinv
  xw = jnp.dot(x_ref[0], w_ref[...], preferred_element_type=jnp.float32)
  y_ref[0] = xw * dinv


def _epilogue_body(agg_ref, y_ref, dinv_ref, b_ref, out_ref):
  out_ref[0] = dinv_ref[0] * (agg_ref[0] + y_ref[0]) + b_ref[...]


def kernel(x, edge_index, W, b):
  n, d, t_window = x.shape
  e = edge_index.shape[1]
  t_per_core = t_window // NC

  unit = NC * NS * K * 3
  e_pad = ((e + unit - 1) // unit) * unit
  n_acc = ((n + NS * 8) // (NS * 8)) * (NS * 8)
  pad = e_pad - e
  src = jnp.concatenate([edge_index[0], jnp.zeros((pad,), jnp.int32)])
  dst = jnp.concatenate([edge_index[1], jnp.full((pad,), n, jnp.int32)])
  src_adj = (src[None, :]
             + (jnp.arange(t_window, dtype=jnp.int32) * n)[:, None]).reshape(-1)

  mesh = plsc.VectorSubcoreMesh(core_axis_name="c", subcore_axis_name="s")

  deg_kernel = pl.kernel(
      functools.partial(_deg_body, e_pad=e_pad, n_acc=n_acc),
      out_type=jax.ShapeDtypeStruct((NC, n_acc, DEG_W), jnp.float32),
      mesh=mesh,
      scratch_types=[
          pltpu.VMEM((K,), jnp.int32),
          pltpu.VMEM((K, DEG_W), jnp.float32),
          pltpu.VMEM_SHARED((n_acc, DEG_W), jnp.float32),
          pltpu.SemaphoreType.DMA,
      ],
  )
  deg_parts = deg_kernel(dst)

  xt = x.transpose(2, 0, 1).reshape(t_window * n, d)
  bn = 400
  nb = n // bn
  y3d, dinv = pl.pallas_call(
      _matmul_body,
      grid=(t_window, nb),
      in_specs=[
          pl.BlockSpec((1, bn, d), lambda t, i: (t, i, 0)),
          pl.BlockSpec((d, d), lambda t, i: (0, 0)),
          pl.BlockSpec((NC, bn, DEG_W), lambda t, i: (0, i, 0)),
      ],
      out_specs=[
          pl.BlockSpec((1, bn, d), lambda t, i: (t, i, 0)),
          pl.BlockSpec((1, bn, 1), lambda t, i: (t, i, 0)),
      ],
      out_shape=[
          jax.ShapeDtypeStruct((t_window, n, d), jnp.float32),
          jax.ShapeDtypeStruct((t_window, n, 1), jnp.float32),
      ],
  )(xt.reshape(t_window, n, d), W, deg_parts)
  y_flat = y3d.reshape(t_window * n, d)

  agg_kernel = pl.kernel(
      functools.partial(_agg_body, e_pad=e_pad, n_acc=n_acc, d=d,
                        t_per_core=t_per_core),
      out_type=jax.ShapeDtypeStruct((t_window, n_acc, d), jnp.float32),
      mesh=mesh,
      scratch_types=[
          [pltpu.VMEM((K,), jnp.int32) for _ in range(3)],
          [pltpu.VMEM((K,), jnp.int32) for _ in range(3)],
          [pltpu.VMEM((K, d), jnp.float32) for _ in range(3)],
          pltpu.VMEM_SHARED((n_acc, d), jnp.float32),
          pltpu.SemaphoreType.DMA,
      ],
  )
  agg = agg_kernel(y_flat, src_adj, dst)

  out = pl.pallas_call(
      _epilogue_body,
      grid=(t_window, nb),
      in_specs=[
          pl.BlockSpec((1, bn, d), lambda t, i: (t, i, 0)),
          pl.BlockSpec((1, bn, d), lambda t, i: (t, i, 0)),
          pl.BlockSpec((1, bn, 1), lambda t, i: (t, i, 0)),
          pl.BlockSpec((1, d), lambda t, i: (0, 0)),
      ],
      out_specs=pl.BlockSpec((1, bn, d), lambda t, i: (t, i, 0)),
      out_shape=jax.ShapeDtypeStruct((t_window, n, d), jnp.float32),
  )(agg, y3d, dinv, b.reshape(1, d))
  return out.transpose(1, 2, 0)
